# baseline (speedup 1.0000x reference)
import jax
import jax.numpy as jnp
from jax import lax
from jax.experimental import pallas as pl
from jax.experimental.pallas import tpu as pltpu
from jax.experimental.pallas import tpu_sc as plsc

VOCAB = 1000
EMBED_DIM = 64
OUT_DIM = 1000
BATCH = 16384

_NC = 2
_NS = 16
_NW = _NC * _NS
_CPW = 32
_BCH = 1024
_NBCH = BATCH // _BCH
_LANES = 16


def _mlp_table_t_body(emb_ref, w0, b0, w1, b1, w2, b2, w3, b3, w4, b4,
                      out_ref):
    ht = jnp.maximum(emb_ref[...], 0.0)
    g = None
    for w, b in ((w0, b0), (w1, b1), (w2, b2), (w3, b3), (w4, b4)):
        if g is None:
            g = lax.dot_general(w[...], ht, (((0,), (1,)), ((), ())),
                                preferred_element_type=jnp.float32)
        else:
            g = lax.dot_general(w[...], g, (((0,), (0,)), ((), ())),
                                preferred_element_type=jnp.float32)
        g = jnp.maximum(g + b[...], 0.0)
    out_ref[...] = g


def _mlp_table_t(emb, ws, bs):
    args = [emb]
    for w, b in zip(ws, bs):
        args += [w, b.reshape(-1, 1)]
    return pl.pallas_call(
        _mlp_table_t_body,
        out_shape=jax.ShapeDtypeStruct((OUT_DIM, VOCAB), jnp.float32),
    )(*args)


def _zgather_body(x_ref, tab_ref, z_ref, x_v, tt_v, z_v, zsem0, zsem1):
    wid = lax.axis_index("s") * _NC + lax.axis_index("c")
    c0 = jnp.where(wid == _NW - 1, OUT_DIM - _CPW, wid * _CPW)
    pltpu.sync_copy(x_ref, x_v)
    pltpu.sync_copy(tab_ref.at[pl.ds(c0, _CPW)], tt_v)
    zsems = (zsem0, zsem1)
    zc = [None, None]
    for blk in range(_NBCH):
        buf = blk % 2
        if zc[buf] is not None:
            zc[buf].wait()

        def body(k, _):
            xv = x_v[pl.ds(blk * _BCH + k * _LANES, _LANES)]
            for c in range(_CPW):
                row = jnp.full((_LANES,), c, jnp.int32)
                vals = plsc.load_gather(tt_v, [row, xv])
                z_v[buf, c, pl.ds(k * _LANES, _LANES)] = vals
            return 0

        lax.fori_loop(0, _BCH // _LANES, body, 0)
        zc[buf] = pltpu.async_copy(
            z_v.at[buf],
            z_ref.at[pl.ds(c0, _CPW), pl.ds(blk * _BCH, _BCH)],
            zsems[buf])
    zc[0].wait()
    zc[1].wait()


def _zgather(x, table_t):
    return pl.kernel(
        _zgather_body,
        out_type=jax.ShapeDtypeStruct((OUT_DIM, BATCH), jnp.float32),
        mesh=plsc.VectorSubcoreMesh(core_axis_name="c", subcore_axis_name="s"),
        compiler_params=pltpu.CompilerParams(needs_layout_passes=False),
        scratch_types=[
            pltpu.VMEM((BATCH,), jnp.int32),
            pltpu.VMEM((_CPW, VOCAB), jnp.float32),
            pltpu.VMEM((2, _CPW, _BCH), jnp.float32),
            pltpu.SemaphoreType.DMA,
            pltpu.SemaphoreType.DMA,
        ],
    )(x, table_t)


def kernel(x, emb, W0, b0, W1, b1, W2, b2, W3, b3, W4, b4):
    table_t = _mlp_table_t(emb, (W0, W1, W2, W3, W4), (b0, b1, b2, b3, b4))
    z = _zgather(x, table_t)
    return z.T
